# Initial kernel scaffold; baseline (speedup 1.0000x reference)
import jax
import jax.numpy as jnp
from jax.experimental import pallas as pl


def kernel(logits, codebook):
    raise NotImplementedError("write your pallas kernel here")


# speedup vs baseline: 1.2337x; 1.2337x over previous
import functools

import jax
import jax.numpy as jnp
from jax import lax
from jax.experimental import pallas as pl
from jax.experimental.pallas import tpu as pltpu
from jax.experimental.pallas import tpu_sc as plsc

NUM_CODES = 8192
CODE_DIM = 32
ROWS_PER_BLOCK = 128

SC_CORES = 2
SC_SUBCORES = 16
SC_WORKERS = SC_CORES * SC_SUBCORES


def _argmax_onehot_body(lg_ref, g_ref, y_ref, idx_ref):
    m = lg_ref[...] + g_ref[...]
    mx = jnp.max(m, axis=1, keepdims=True)
    col = lax.broadcasted_iota(jnp.int32, m.shape, 1)
    idx = jnp.min(jnp.where(m == mx, col, NUM_CODES), axis=1).astype(jnp.int32)
    y_ref[...] = (col == idx[:, None]).astype(jnp.float32)
    idx_ref[0, 0, :] = idx


def _make_sc_gather(n_rows):
    b_per_w = n_rows // SC_WORKERS
    n_chunks = -(-b_per_w // 128)
    chunk = b_per_w // n_chunks
    assert chunk * n_chunks == b_per_w and chunk % 8 == 0

    mesh = plsc.VectorSubcoreMesh(
        core_axis_name="c", subcore_axis_name="s", num_cores=SC_CORES,
        num_subcores=SC_SUBCORES)

    @functools.partial(
        pl.kernel,
        out_type=jax.ShapeDtypeStruct((n_rows, CODE_DIM), jnp.float32),
        mesh=mesh,
        scratch_types=[
            pltpu.VMEM((b_per_w,), jnp.int32),
            pltpu.VMEM((b_per_w, CODE_DIM), jnp.float32),
            pltpu.SemaphoreType.DMA,
        ],
        compiler_params=pltpu.CompilerParams(use_tc_tiling_on_sc=False),
    )
    def sc_gather(codebook_hbm, idx_hbm, z_hbm, idx_v, rows_v, sem):
        wid = lax.axis_index("s") * SC_CORES + lax.axis_index("c")
        base = wid * b_per_w
        pltpu.sync_copy(idx_hbm.at[pl.ds(base, b_per_w)], idx_v)
        for j in range(n_chunks):
            pltpu.async_copy(
                codebook_hbm.at[idx_v.at[pl.ds(j * chunk, chunk)]],
                rows_v.at[pl.ds(j * chunk, chunk)],
                sem,
            ).wait()
        pltpu.sync_copy(rows_v, z_hbm.at[pl.ds(base, b_per_w)])

    return sc_gather


def kernel(logits, codebook):
    B, T, N = logits.shape
    R = B * T
    g = jax.random.gumbel(jax.random.key(1), logits.shape, logits.dtype)
    lg2 = logits.reshape(R, N)
    g2 = g.reshape(R, N)
    nblk = R // ROWS_PER_BLOCK

    y2, idx3 = pl.pallas_call(
        _argmax_onehot_body,
        grid=(nblk,),
        in_specs=[
            pl.BlockSpec((ROWS_PER_BLOCK, N), lambda i: (i, 0)),
            pl.BlockSpec((ROWS_PER_BLOCK, N), lambda i: (i, 0)),
        ],
        out_specs=[
            pl.BlockSpec((ROWS_PER_BLOCK, N), lambda i: (i, 0)),
            pl.BlockSpec((1, 1, ROWS_PER_BLOCK), lambda i: (i, 0, 0)),
        ],
        out_shape=[
            jax.ShapeDtypeStruct((R, N), jnp.float32),
            jax.ShapeDtypeStruct((nblk, 1, ROWS_PER_BLOCK), jnp.int32),
        ],
        compiler_params=pltpu.CompilerParams(
            dimension_semantics=("arbitrary",),
        ),
    )(lg2, g2)

    idx = idx3.reshape(R)
    z2 = _make_sc_gather(R)(codebook, idx)
    return z2.reshape(B, T, CODE_DIM), y2.reshape(B, T, N)
